# baseline (speedup 1.0000x reference)
import functools

import jax
import jax.numpy as jnp
from jax import lax
from jax.experimental import pallas as pl
from jax.experimental.pallas import tpu as pltpu
from jax.experimental.pallas import tpu_sc as plsc

_N = 10000
_E = 320000
_D = 128
_T = 8
_G = 384

_NC = 2
_NS = 16
_NW = _NC * _NS

_CHUNK = 128
_GRP = 8
_CPW = _GRP * (-(-_E // (_NW * _CHUNK * _GRP)))
_NG = _CPW // _GRP
_EP = _NW * _CPW * _CHUNK
_NP = 10240
_RPT = _NP // _NS


def _spmm_body(idx_hbm, table_hbm, zeros_hbm, out_hbm,
               idx_v, buf_v, acc_sh, isem, ssem):
    cid = lax.axis_index("c")
    sid = lax.axis_index("s")
    wid = sid * _NC + cid
    r0 = sid * _RPT
    pltpu.sync_copy(zeros_hbm, acc_sh.at[pl.ds(r0, _RPT)])
    pltpu.sync_copy(idx_hbm.at[wid, 0], idx_v.at[0])

    @pl.when(_NG > 1)
    def _():
        pltpu.async_copy(idx_hbm.at[wid, 1], idx_v.at[1], isem.at[1])

    plsc.subcore_barrier()

    def group(g, carry):
        p = lax.rem(g, 2)

        @pl.when(g > 0)
        def _():
            pltpu.make_async_copy(idx_hbm.at[wid, g], idx_v.at[p],
                                  isem.at[p]).wait()

        sdescs = []
        for k in range(_GRP):
            b = k % 2
            if k >= 2:
                sdescs[k - 2].wait()
            pltpu.sync_copy(table_hbm.at[idx_v.at[p, k, 0]], buf_v.at[b])
            sdescs.append(
                pltpu.async_copy(buf_v.at[b], acc_sh.at[idx_v.at[p, k, 1]],
                                 ssem.at[b], add=True))
        sdescs[_GRP - 2].wait()
        sdescs[_GRP - 1].wait()

        @pl.when(g + 2 < _NG)
        def _():
            pltpu.async_copy(idx_hbm.at[wid, g + 2], idx_v.at[p], isem.at[p])

        return carry

    lax.fori_loop(0, _NG, group, 0)
    plsc.subcore_barrier()
    pltpu.sync_copy(acc_sh.at[pl.ds(r0, _RPT)],
                    out_hbm.at[cid, pl.ds(r0, _RPT)])


_spmm = pl.kernel(
    _spmm_body,
    out_type=jax.ShapeDtypeStruct((_NC, _NP, _D), jnp.float32),
    mesh=plsc.VectorSubcoreMesh(core_axis_name="c", subcore_axis_name="s",
                                num_cores=_NC, num_subcores=_NS),
    scratch_types=[
        pltpu.VMEM((2, _GRP, 2, _CHUNK), jnp.int32),
        pltpu.VMEM((2, _CHUNK, _D), jnp.float32),
        pltpu.VMEM_SHARED((_NP, _D), jnp.float32),
        pltpu.SemaphoreType.DMA((2,)),
        pltpu.SemaphoreType.DMA((2,)),
    ],
)


def _combine_deg_body(dout_ref, din_ref, cs_ref, rs_ref):
    d_out = dout_ref[0, :, 0:1] + dout_ref[1, :, 0:1]
    d_in = din_ref[0, :, 0:1] + din_ref[1, :, 0:1]
    cs_ref[...] = lax.rsqrt(jnp.maximum(d_out, 1.0))
    rs_ref[...] = lax.rsqrt(jnp.maximum(d_in, 1.0))


def _scale_body(x_ref, cs_ref, out_ref):
    out_ref[0] = x_ref[0] * cs_ref[...]


def _preact_body(p_ref, rs_ref, w_ref, b_ref, out_ref):
    m = (p_ref[0] + p_ref[1]) * rs_ref[...]
    out_ref[...] = jnp.dot(m, w_ref[...],
                           preferred_element_type=jnp.float32) + b_ref[...]


def _gates_body(p_ref, rs_ref, it_ref, hp_ref, cs_ref, w_ref, b_ref,
                h_ref, hs_ref):
    m = (p_ref[0] + p_ref[1]) * rs_ref[...]
    hg = jnp.dot(m, w_ref[...],
                 preferred_element_type=jnp.float32) + b_ref[...]
    it = it_ref[...]
    r = jax.nn.sigmoid(it[:, 0:_D] + hg[:, 0:_D])
    z = jax.nn.sigmoid(it[:, _D:2 * _D] + hg[:, _D:2 * _D])
    n = jnp.tanh(it[:, 2 * _D:] + r * hg[:, 2 * _D:])
    h = (1.0 - z) * n + z * hp_ref[...]
    h_ref[...] = h
    hs_ref[...] = h * cs_ref[...]


_NB = 2048
_NBLK = _NP // _NB


def _combine_deg(dout_p, din_p):
    return pl.pallas_call(
        _combine_deg_body,
        grid=(_NBLK,),
        in_specs=[
            pl.BlockSpec((_NC, _NB, _D), lambda i: (0, i, 0)),
            pl.BlockSpec((_NC, _NB, _D), lambda i: (0, i, 0)),
        ],
        out_specs=[
            pl.BlockSpec((_NB, 1), lambda i: (i, 0)),
            pl.BlockSpec((_NB, 1), lambda i: (i, 0)),
        ],
        out_shape=[
            jax.ShapeDtypeStruct((_NP, 1), jnp.float32),
            jax.ShapeDtypeStruct((_NP, 1), jnp.float32),
        ],
    )(dout_p, din_p)


def _scale(xp, cs):
    return pl.pallas_call(
        _scale_body,
        grid=(_T, _NBLK),
        in_specs=[
            pl.BlockSpec((1, _NB, _D), lambda t, i: (t, i, 0)),
            pl.BlockSpec((_NB, 1), lambda t, i: (i, 0)),
        ],
        out_specs=pl.BlockSpec((1, _NB, _D), lambda t, i: (t, i, 0)),
        out_shape=jax.ShapeDtypeStruct((_T, _NP, _D), jnp.float32),
    )(xp, cs)


def _preact(p, rs, w, b):
    return pl.pallas_call(
        _preact_body,
        grid=(_NBLK,),
        in_specs=[
            pl.BlockSpec((_NC, _NB, _D), lambda i: (0, i, 0)),
            pl.BlockSpec((_NB, 1), lambda i: (i, 0)),
            pl.BlockSpec((_D, _G), lambda i: (0, 0)),
            pl.BlockSpec((1, _G), lambda i: (0, 0)),
        ],
        out_specs=pl.BlockSpec((_NB, _G), lambda i: (i, 0)),
        out_shape=jax.ShapeDtypeStruct((_NP, _G), jnp.float32),
    )(p, rs, w, b)


def _gates(p, rs, it, hp, cs, w, b):
    return pl.pallas_call(
        _gates_body,
        grid=(_NBLK,),
        in_specs=[
            pl.BlockSpec((_NC, _NB, _D), lambda i: (0, i, 0)),
            pl.BlockSpec((_NB, 1), lambda i: (i, 0)),
            pl.BlockSpec((_NB, _G), lambda i: (i, 0)),
            pl.BlockSpec((_NB, _D), lambda i: (i, 0)),
            pl.BlockSpec((_NB, 1), lambda i: (i, 0)),
            pl.BlockSpec((_D, _G), lambda i: (0, 0)),
            pl.BlockSpec((1, _G), lambda i: (0, 0)),
        ],
        out_specs=[
            pl.BlockSpec((_NB, _D), lambda i: (i, 0)),
            pl.BlockSpec((_NB, _D), lambda i: (i, 0)),
        ],
        out_shape=[
            jax.ShapeDtypeStruct((_NP, _D), jnp.float32),
            jax.ShapeDtypeStruct((_NP, _D), jnp.float32),
        ],
    )(p, rs, it, hp, cs, w, b)


def kernel(x, edge_index, Wi1, bi1, Wh1, bh1, Wi2, bi2, Wh2, bh2):
    src = edge_index[0]
    dst = edge_index[1]
    fill = jnp.full((_EP - _E,), _N, jnp.int32)
    src_p = jnp.concatenate([src, fill]).reshape(_NW, _NG, _GRP, _CHUNK)
    dst_p = jnp.concatenate([dst, fill]).reshape(_NW, _NG, _GRP, _CHUNK)
    idx_fwd = jnp.stack([src_p, dst_p], axis=3)
    idx_rev = jnp.stack([dst_p, src_p], axis=3)
    zeros_t = jnp.zeros((_RPT, _D), jnp.float32)
    ones_t = jnp.ones((_NP, _D), jnp.float32)
    zpart = jnp.zeros((_NC, _NP, _D), jnp.float32)

    dout_p = _spmm(idx_rev, ones_t, zeros_t)
    din_p = _spmm(idx_fwd, ones_t, zeros_t)
    cs, rs = _combine_deg(dout_p, din_p)

    xp = jnp.pad(jnp.transpose(x, (1, 0, 2)), ((0, 0), (0, _NP - _N), (0, 0)))
    xs = _scale(xp, cs)

    feats = [xs[t] for t in range(_T)]
    params = [(Wi1, bi1, Wh1, bh1), (Wi2, bi2, Wh2, bh2)]
    outs = None
    for (Wi, bi, Wh, bh) in params:
        bi2d = bi.reshape(1, _G)
        bh2d = bh.reshape(1, _G)
        preacts = []
        for t in range(_T):
            pfeat = _spmm(idx_fwd, feats[t], zeros_t)
            preacts.append(_preact(pfeat, rs, Wi, bi2d))
        h = jnp.zeros((_NP, _D), jnp.float32)
        hs_prev = None
        outs = []
        new_feats = []
        for t in range(_T):
            ph = zpart if t == 0 else _spmm(idx_fwd, hs_prev, zeros_t)
            h, hs_prev = _gates(ph, rs, preacts[t], h, cs, Wh, bh2d)
            outs.append(h)
            new_feats.append(hs_prev)
        feats = new_feats
    return jnp.stack(outs, axis=1)[:_N]


# speedup vs baseline: 1.3999x; 1.3999x over previous
import jax
import jax.numpy as jnp
from jax import lax
from jax.experimental import pallas as pl
from jax.experimental.pallas import tpu as pltpu
from jax.experimental.pallas import tpu_sc as plsc

_N = 10000
_E = 320000
_D = 128
_T = 8
_G = 384

_NC = 2
_NS = 16
_NW = _NC * _NS

_CHUNK = 128
_CPW = -(-_E // (_NW * _CHUNK))
_EP = _NW * _CPW * _CHUNK
_EPW = _CPW * _CHUNK
_NP = 10240
_RPT = _NP // _NS
_LANES = 16


def _spmm_body(gidx_hbm, sidx_hbm, table_hbm, zeros_hbm, out_hbm,
               gidx_v, sidx_v, buf_v, acc_sh, sem):
    cid = lax.axis_index("c")
    sid = lax.axis_index("s")
    wid = sid * _NC + cid
    r0 = sid * _RPT
    pltpu.sync_copy(zeros_hbm, acc_sh.at[pl.ds(r0, _RPT)])
    plsc.subcore_barrier()
    pltpu.sync_copy(gidx_hbm.at[wid], gidx_v)
    pltpu.sync_copy(sidx_hbm.at[wid], sidx_v)

    def step(j, carry):
        pltpu.async_copy(table_hbm.at[gidx_v.at[j]], buf_v, sem).wait()
        pltpu.sync_copy(buf_v, acc_sh.at[sidx_v.at[j]], add=True)
        return carry

    lax.fori_loop(0, _CPW, step, 0)
    plsc.subcore_barrier()
    pltpu.sync_copy(acc_sh.at[pl.ds(r0, _RPT)],
                    out_hbm.at[cid, pl.ds(r0, _RPT)])


_spmm = pl.kernel(
    _spmm_body,
    out_type=jax.ShapeDtypeStruct((_NC, _NP, _D), jnp.float32),
    mesh=plsc.VectorSubcoreMesh(core_axis_name="c", subcore_axis_name="s",
                                num_cores=_NC, num_subcores=_NS),
    scratch_types=[
        pltpu.VMEM((_CPW, _CHUNK), jnp.int32),
        pltpu.VMEM((_CPW, _CHUNK), jnp.int32),
        pltpu.VMEM((_CHUNK, _D), jnp.float32),
        pltpu.VMEM_SHARED((_NP, _D), jnp.float32),
        pltpu.SemaphoreType.DMA,
    ],
)


def _combine_deg_body(do_ref, di_ref, cs_ref, rs_ref):
    d_out = do_ref[0, :, 0:1] + do_ref[1, :, 0:1]
    d_in = di_ref[0, :, 0:1] + di_ref[1, :, 0:1]
    cs_ref[...] = lax.rsqrt(jnp.maximum(d_out, 1.0))
    rs_ref[...] = lax.rsqrt(jnp.maximum(d_in, 1.0))


def _scale_body(x_ref, cs_ref, out_ref):
    out_ref[0] = x_ref[0] * cs_ref[...]


def _preact_body(p_ref, rs_ref, w_ref, b_ref, out_ref):
    m = (p_ref[0] + p_ref[1]) * rs_ref[...]
    out_ref[...] = jnp.dot(m, w_ref[...],
                           preferred_element_type=jnp.float32) + b_ref[...]


def _gates_body(p_ref, rs_ref, it_ref, hp_ref, cs_ref, w_ref, b_ref,
                h_ref, hs_ref):
    m = (p_ref[0] + p_ref[1]) * rs_ref[...]
    hg = jnp.dot(m, w_ref[...],
                 preferred_element_type=jnp.float32) + b_ref[...]
    it = it_ref[...]
    r = jax.nn.sigmoid(it[:, 0:_D] + hg[:, 0:_D])
    z = jax.nn.sigmoid(it[:, _D:2 * _D] + hg[:, _D:2 * _D])
    n = jnp.tanh(it[:, 2 * _D:] + r * hg[:, 2 * _D:])
    h = (1.0 - z) * n + z * hp_ref[...]
    h_ref[...] = h
    hs_ref[...] = h * cs_ref[...]


_NB = 2048
_NBLK = _NP // _NB


def _combine_deg(dout_p, din_p):
    return pl.pallas_call(
        _combine_deg_body,
        grid=(_NBLK,),
        in_specs=[
            pl.BlockSpec((_NC, _NB, _D), lambda i: (0, i, 0)),
            pl.BlockSpec((_NC, _NB, _D), lambda i: (0, i, 0)),
        ],
        out_specs=[
            pl.BlockSpec((_NB, 1), lambda i: (i, 0)),
            pl.BlockSpec((_NB, 1), lambda i: (i, 0)),
        ],
        out_shape=[
            jax.ShapeDtypeStruct((_NP, 1), jnp.float32),
            jax.ShapeDtypeStruct((_NP, 1), jnp.float32),
        ],
    )(dout_p, din_p)


def _scale(xp, cs):
    return pl.pallas_call(
        _scale_body,
        grid=(_T, _NBLK),
        in_specs=[
            pl.BlockSpec((1, _NB, _D), lambda t, i: (t, i, 0)),
            pl.BlockSpec((_NB, 1), lambda t, i: (i, 0)),
        ],
        out_specs=pl.BlockSpec((1, _NB, _D), lambda t, i: (t, i, 0)),
        out_shape=jax.ShapeDtypeStruct((_T, _NP, _D), jnp.float32),
    )(xp, cs)


def _preact(p, rs, w, b):
    return pl.pallas_call(
        _preact_body,
        grid=(_NBLK,),
        in_specs=[
            pl.BlockSpec((_NC, _NB, _D), lambda i: (0, i, 0)),
            pl.BlockSpec((_NB, 1), lambda i: (i, 0)),
            pl.BlockSpec((_D, _G), lambda i: (0, 0)),
            pl.BlockSpec((1, _G), lambda i: (0, 0)),
        ],
        out_specs=pl.BlockSpec((_NB, _G), lambda i: (i, 0)),
        out_shape=jax.ShapeDtypeStruct((_NP, _G), jnp.float32),
    )(p, rs, w, b)


def _gates(p, rs, it, hp, cs, w, b):
    return pl.pallas_call(
        _gates_body,
        grid=(_NBLK,),
        in_specs=[
            pl.BlockSpec((_NC, _NB, _D), lambda i: (0, i, 0)),
            pl.BlockSpec((_NB, 1), lambda i: (i, 0)),
            pl.BlockSpec((_NB, _G), lambda i: (i, 0)),
            pl.BlockSpec((_NB, _D), lambda i: (i, 0)),
            pl.BlockSpec((_NB, 1), lambda i: (i, 0)),
            pl.BlockSpec((_D, _G), lambda i: (0, 0)),
            pl.BlockSpec((1, _G), lambda i: (0, 0)),
        ],
        out_specs=[
            pl.BlockSpec((_NB, _D), lambda i: (i, 0)),
            pl.BlockSpec((_NB, _D), lambda i: (i, 0)),
        ],
        out_shape=[
            jax.ShapeDtypeStruct((_NP, _D), jnp.float32),
            jax.ShapeDtypeStruct((_NP, _D), jnp.float32),
        ],
    )(p, rs, it, hp, cs, w, b)


def kernel(x, edge_index, Wi1, bi1, Wh1, bh1, Wi2, bi2, Wh2, bh2):
    src = edge_index[0]
    dst = edge_index[1]
    fill = jnp.full((_EP - _E,), _N, jnp.int32)
    src_p = jnp.concatenate([src, fill]).reshape(_NW, _CPW, _CHUNK)
    dst_p = jnp.concatenate([dst, fill]).reshape(_NW, _CPW, _CHUNK)
    zeros_t = jnp.zeros((_RPT, _D), jnp.float32)
    ones_t = jnp.ones((_NP, _D), jnp.float32)
    zpart = jnp.zeros((_NC, _NP, _D), jnp.float32)

    dout_p = _spmm(dst_p, src_p, ones_t, zeros_t)
    din_p = _spmm(src_p, dst_p, ones_t, zeros_t)
    cs, rs = _combine_deg(dout_p, din_p)

    xp = jnp.pad(jnp.transpose(x, (1, 0, 2)), ((0, 0), (0, _NP - _N), (0, 0)))
    xs = _scale(xp, cs)

    feats = [xs[t] for t in range(_T)]
    params = [(Wi1, bi1, Wh1, bh1), (Wi2, bi2, Wh2, bh2)]
    outs = None
    for (Wi, bi, Wh, bh) in params:
        bi2d = bi.reshape(1, _G)
        bh2d = bh.reshape(1, _G)
        preacts = []
        for t in range(_T):
            pfeat = _spmm(src_p, dst_p, feats[t], zeros_t)
            preacts.append(_preact(pfeat, rs, Wi, bi2d))
        h = jnp.zeros((_NP, _D), jnp.float32)
        hs_prev = None
        outs = []
        new_feats = []
        for t in range(_T):
            ph = zpart if t == 0 else _spmm(src_p, dst_p, hs_prev, zeros_t)
            h, hs_prev = _gates(ph, rs, preacts[t], h, cs, Wh, bh2d)
            outs.append(h)
            new_feats.append(hs_prev)
        feats = new_feats
    return jnp.stack(outs, axis=1)[:_N]
